# baseline (speedup 1.0000x reference)
import functools

import jax
import jax.numpy as jnp
from jax import lax
from jax.experimental import pallas as pl
from jax.experimental.pallas import tpu as pltpu
from jax.experimental.pallas import tpu_sc as plsc

NC = 2
NS = 16
NW = NC * NS

B = 16384 * 50
D = 32
BPW = B // NW
C = 128
NCHUNK = BPW // C
NITER = NCHUNK // 2

_mesh = plsc.VectorSubcoreMesh(core_axis_name="c", subcore_axis_name="s")


@functools.partial(
    pl.kernel,
    mesh=_mesh,
    out_type=jax.ShapeDtypeStruct((B, 128), jnp.float32),
    scratch_types=[
        pltpu.VMEM((2, 1, 128), jnp.int32),
        pltpu.VMEM((2, 128), jnp.int32),
        pltpu.VMEM((2, C, 128), jnp.float32),
        pltpu.SemaphoreType.DMA,
        pltpu.SemaphoreType.DMA,
        pltpu.SemaphoreType.DMA,
        pltpu.SemaphoreType.DMA,
        pltpu.SemaphoreType.DMA,
        pltpu.SemaphoreType.DMA,
    ],
)
def _gather_kernel(idx_hbm, table_hbm, out_hbm,
                   idxc, idxa, aw,
                   si0, si1, sg0, sg1, so0, so1):
    wid = lax.axis_index("s") * NC + lax.axis_index("c")
    idx_row0 = wid * (BPW // 128)
    out_row0 = wid * BPW
    si = (si0, si1)
    sg = (sg0, sg1)
    so = (so0, so1)

    def idx_load(g, b):
        return pltpu.async_copy(
            idx_hbm.at[pl.ds(idx_row0 + g, 1)], idxc.at[b], si[b])

    idx_load(0, 0)
    idx_load(1, 1)

    def body(i, carry):
        for b in range(2):
            g = 2 * i + b
            pltpu.make_async_copy(
                idx_hbm.at[pl.ds(0, 1)], idxc.at[b], si[b]).wait()
            for k in range(8):
                e = idxc[b, 0, pl.ds(k * 16, 16)]
                idxa[b, pl.ds(k * 16, 16)] = e >> 2

            @pl.when(g >= 2)
            def _wait_prev_store():
                pltpu.make_async_copy(
                    aw.at[b], out_hbm.at[pl.ds(0, C)], so[b]).wait()

            pltpu.async_copy(table_hbm.at[idxa.at[b]], aw.at[b], sg[b])

            @pl.when(i < NITER - 1)
            def _next_idx_load():
                idx_load(g + 2, b)

        for b in range(2):
            g = 2 * i + b
            pltpu.make_async_copy(
                table_hbm.at[idxa.at[b]], aw.at[b], sg[b]).wait()
            pltpu.async_copy(
                aw.at[b], out_hbm.at[pl.ds(out_row0 + g * C, C)], so[b])
        return carry

    lax.fori_loop(0, NITER, body, 0)
    for b in range(2):
        pltpu.make_async_copy(
            aw.at[b], out_hbm.at[pl.ds(0, C)], so[b]).wait()


def kernel(word_ids, embedding_weight):
    zero = jnp.min(word_ids).astype(jnp.int32) >> 31
    wids = word_ids.astype(jnp.int32) ^ zero
    idx2 = wids.reshape(B // 128, 128)
    tbl2 = embedding_weight.reshape(-1, 128)
    wide = _gather_kernel(idx2, tbl2)
    w4 = wide.reshape(16384, 50, 4, D)
    r = (wids & 3)[:, :, None]
    out = jnp.where(
        r == 0, w4[:, :, 0, :],
        jnp.where(r == 1, w4[:, :, 1, :],
                  jnp.where(r == 2, w4[:, :, 2, :], w4[:, :, 3, :])))
    return out


# speedup vs baseline: 1.6637x; 1.6637x over previous
import functools

import jax
import jax.numpy as jnp
from jax import lax
from jax.experimental import pallas as pl
from jax.experimental.pallas import tpu as pltpu
from jax.experimental.pallas import tpu_sc as plsc

NC = 2
NS = 16
NW = NC * NS

B = 16384 * 50
V = 1000000
D = 32
BPW = B // NW
IRPW = BPW // D
C = 1024
NCHUNK = BPW // C

_mesh = plsc.VectorSubcoreMesh(core_axis_name="c", subcore_axis_name="s")


@functools.partial(
    pl.kernel,
    mesh=_mesh,
    out_type=jax.ShapeDtypeStruct((B, D), jnp.int32),
    compiler_params=pltpu.CompilerParams(use_tc_tiling_on_sc=False),
    scratch_types=[
        pltpu.VMEM((IRPW, D), jnp.int32),
        pltpu.VMEM((BPW,), jnp.int32),
        pltpu.VMEM((2, C, D), jnp.int32),
        pltpu.SemaphoreType.DMA,
        pltpu.SemaphoreType.DMA,
        pltpu.SemaphoreType.DMA,
        pltpu.SemaphoreType.DMA,
    ],
)
def _gather_kernel(comb_hbm, out_hbm, idxc, idxf, rows_v, sg0, sg1, so0, so1):
    wid = lax.axis_index("s") * NC + lax.axis_index("c")
    base = wid * BPW
    pltpu.sync_copy(comb_hbm.at[pl.ds(V + wid * IRPW, IRPW)], idxc)
    for r in range(IRPW):
        for c in range(0, D, 16):
            idxf[pl.ds(r * D + c, 16)] = idxc[r, pl.ds(c, 16)]

    sg = (sg0, sg1)
    so = (so0, so1)
    gather_cp = [None] * NCHUNK
    store_cp = [None] * NCHUNK
    for g in range(NCHUNK):
        b = g % 2
        if g >= 2:
            store_cp[g - 2].wait()
        gather_cp[g] = pltpu.async_copy(
            comb_hbm.at[idxf.at[pl.ds(g * C, C)]], rows_v.at[b], sg[b])
        if g >= 1:
            pb = (g - 1) % 2
            gather_cp[g - 1].wait()
            store_cp[g - 1] = pltpu.async_copy(
                rows_v.at[pb], out_hbm.at[pl.ds(base + (g - 1) * C, C)], so[pb])
    gather_cp[NCHUNK - 1].wait()
    lb = (NCHUNK - 1) % 2
    store_cp[NCHUNK - 1] = pltpu.async_copy(
        rows_v.at[lb], out_hbm.at[pl.ds(base + (NCHUNK - 1) * C, C)], so[lb])
    store_cp[NCHUNK - 2].wait()
    store_cp[NCHUNK - 1].wait()


def kernel(word_ids, embedding_weight):
    tbl_i = lax.bitcast_convert_type(embedding_weight, jnp.int32)
    idx_i = word_ids.astype(jnp.int32).reshape(B // D, D)
    comb = jnp.concatenate([tbl_i, idx_i], axis=0)
    out_i = _gather_kernel(comb)
    out = lax.bitcast_convert_type(out_i, jnp.float32)
    return out.reshape(word_ids.shape + (embedding_weight.shape[1],))


# speedup vs baseline: 2.3942x; 1.4390x over previous
import functools

import jax
import jax.numpy as jnp
from jax import lax
from jax.experimental import pallas as pl
from jax.experimental.pallas import tpu as pltpu
from jax.experimental.pallas import tpu_sc as plsc

NC = 2
NS = 16
NW = NC * NS

B = 16384 * 50
D = 32
BPW = B // NW
C = 1600
NCHUNK = BPW // C

_mesh = plsc.VectorSubcoreMesh(core_axis_name="c", subcore_axis_name="s")


@functools.partial(
    pl.kernel,
    mesh=_mesh,
    out_type=jax.ShapeDtypeStruct((B, D), jnp.float32),
    compiler_params=pltpu.CompilerParams(use_tc_tiling_on_sc=False),
    scratch_types=[
        pltpu.VMEM((BPW,), jnp.int32),
        pltpu.VMEM((2, C, D), jnp.float32),
        pltpu.SemaphoreType.DMA,
        pltpu.SemaphoreType.DMA,
        pltpu.SemaphoreType.DMA,
        pltpu.SemaphoreType.DMA,
    ],
)
def _gather_kernel(idx_hbm, table_hbm, out_hbm, idx_v, rows_v, sg0, sg1, so0, so1):
    wid = lax.axis_index("s") * NC + lax.axis_index("c")
    base = wid * BPW
    pltpu.sync_copy(idx_hbm.at[pl.ds(base, BPW)], idx_v)

    sg = (sg0, sg1)
    so = (so0, so1)
    gather_cp = [None] * NCHUNK
    store_cp = [None] * NCHUNK
    for g in range(NCHUNK):
        b = g % 2
        if g >= 2:
            store_cp[g - 2].wait()
        gather_cp[g] = pltpu.async_copy(
            table_hbm.at[idx_v.at[pl.ds(g * C, C)]], rows_v.at[b], sg[b])
        if g >= 1:
            pb = (g - 1) % 2
            gather_cp[g - 1].wait()
            store_cp[g - 1] = pltpu.async_copy(
                rows_v.at[pb], out_hbm.at[pl.ds(base + (g - 1) * C, C)], so[pb])
    gather_cp[NCHUNK - 1].wait()
    lb = (NCHUNK - 1) % 2
    store_cp[NCHUNK - 1] = pltpu.async_copy(
        rows_v.at[lb], out_hbm.at[pl.ds(base + (NCHUNK - 1) * C, C)], so[lb])
    store_cp[NCHUNK - 2].wait()
    store_cp[NCHUNK - 1].wait()


def kernel(word_ids, embedding_weight):
    idx = word_ids.reshape(-1).astype(jnp.int32)
    out = _gather_kernel(idx, embedding_weight)
    return out.reshape(word_ids.shape + (embedding_weight.shape[1],))
